# Initial kernel scaffold; baseline (speedup 1.0000x reference)
import jax
import jax.numpy as jnp
from jax.experimental import pallas as pl


def kernel(x, edge_index, label, Wi1, asi1, adi1, Wo1, aso1, ado1, Wi2, asi2, adi2, Wo2, aso2, ado2, Wi3, asi3, adi3, Wo3, aso3, ado3, Wm, bm):
    raise NotImplementedError("write your pallas kernel here")


# speedup vs baseline: 41.2887x; 41.2887x over previous
import jax
import jax.numpy as jnp
from jax import lax
from jax.experimental import pallas as pl
from jax.experimental.pallas import tpu as pltpu
from jax.experimental.pallas import tpu_sc as plsc

NN = 10000
EE = 320000
HH = 128
NHEADS = 4
DHEAD = 32

NC = 2
NS = 16
NW = NC * NS
CH = 80
NG = CH // 16
NBLK = EE // CH
BPW = NBLK // NW
NNP = 10240
NPN = NNP // NS

_MESH = plsc.VectorSubcoreMesh(core_axis_name="c", subcore_axis_name="s",
                               num_cores=NC, num_subcores=NS)


def _i16(v):
    return jnp.full((16,), v, jnp.int32)


def _zero_rows(ref, nrows, ncols16):
    z = jnp.zeros((16,), jnp.float32)

    def body(i, _):
        for cc in range(ncols16):
            ref[i, pl.ds(cc * 16, 16)] = z
        return 0

    lax.fori_loop(0, nrows, body, 0)


def _sc_scores(a4, b4, idx_a, idx_b):
    CB = 16
    NCHUNK = (BPW + CB - 1) // CB
    IPW = NCHUNK * CB * CH
    GPW = NCHUNK * CB * CH * NHEADS
    DPW = 40064

    def body(a_hbm, b_hbm, ia_hbm, ib_hbm,
             den_hbm, g_hbm,
             a_v, b_v, ia_v, ib_v, gchunk, den_v):
        c = lax.axis_index("c")
        s = lax.axis_index("s")
        wid = c * NS + s

        z16 = jnp.zeros((16,), jnp.float32)

        def zrow(i, _):
            den_v[pl.ds(i * 16, 16)] = z16
            return 0

        lax.fori_loop(0, DPW // 16, zrow, 0)

        pltpu.sync_copy(a_hbm, a_v)
        pltpu.sync_copy(b_hbm, b_v)
        iota16 = lax.iota(jnp.int32, 16)

        for cc in range(NCHUNK):
            nblk = min(CB, BPW - cc * CB)
            pltpu.sync_copy(
                ia_hbm.at[wid, 0, pl.ds(cc * CB * CH, CB * CH)], ia_v)
            pltpu.sync_copy(
                ib_hbm.at[wid, 0, pl.ds(cc * CB * CH, CB * CH)], ib_v)

            def blk(j2, _):
                for kk in range(NG):
                    src16 = ia_v[pl.ds(j2 * CH + kk * 16, 16)]
                    dst16 = ib_v[pl.ds(j2 * CH + kk * 16, 16)]
                    row16 = iota16 * NHEADS + (j2 * CH * NHEADS +
                                               kk * 16 * NHEADS)
                    for hh in range(NHEADS):
                        ea = plsc.load_gather(a_v, [src16 * NHEADS + hh])
                        eb = plsc.load_gather(b_v, [dst16 * NHEADS + hh])
                        v = ea + eb
                        e = jnp.where(v >= 0.0, v, v * 0.2)
                        g16 = jnp.exp(e)
                        plsc.store_scatter(gchunk, [row16 + hh], g16)
                        plsc.addupdate_scatter(
                            den_v, [dst16 * NHEADS + hh], g16)
                return 0

            lax.fori_loop(0, nblk, blk, 0)
            pltpu.sync_copy(
                gchunk,
                g_hbm.at[wid, 0, pl.ds(cc * CB * CH * NHEADS,
                                       CB * CH * NHEADS)])

        pltpu.sync_copy(den_v, den_hbm.at[c, s, 0])

    f = pl.kernel(
        body,
        compiler_params=pltpu.CompilerParams(needs_layout_passes=False),
        out_type=(pltpu.HBM((NC, NS, 1, DPW), jnp.float32),
                  pltpu.HBM((NW, 1, GPW), jnp.float32)),
        mesh=_MESH,
        scratch_types=[
            pltpu.VMEM((NN * NHEADS,), jnp.float32),
            pltpu.VMEM((NN * NHEADS,), jnp.float32),
            pltpu.VMEM((CB * CH,), jnp.int32),
            pltpu.VMEM((CB * CH,), jnp.int32),
            pltpu.VMEM((CB * CH * NHEADS,), jnp.float32),
            pltpu.VMEM((DPW,), jnp.float32),
        ],
    )
    return f(a4, b4, idx_a, idx_b)


def _sc_alpha(g4, r8, idx_s, off):
    CB = 16
    NCHUNK = (BPW + CB - 1) // CB

    def body(g_hbm, r_hbm, is_hbm, al_hbm, is_v, r_v, gchunk):
        c = lax.axis_index("c")
        s = lax.axis_index("s")
        wid = c * NS + s
        iota16 = lax.iota(jnp.int32, 16)

        pltpu.sync_copy(r_hbm, r_v)
        for cc in range(NCHUNK):
            nblk = min(CB, BPW - cc * CB)
            pltpu.sync_copy(
                is_hbm.at[wid, 0, pl.ds(cc * CB * CH, CB * CH)], is_v)
            pltpu.sync_copy(
                g_hbm.at[wid, 0, pl.ds(cc * CB * CH * NHEADS,
                                       CB * CH * NHEADS)], gchunk)

            def blk(j2, _):
                for kk in range(NG):
                    dst16 = is_v[pl.ds(j2 * CH + kk * 16, 16)]
                    row16 = iota16 * NHEADS + (j2 * CH * NHEADS +
                                               kk * 16 * NHEADS)
                    for hh in range(NHEADS):
                        g16 = plsc.load_gather(gchunk, [row16 + hh])
                        r16 = plsc.load_gather(
                            r_v, [dst16 * (2 * NHEADS) + (off + hh)])
                        plsc.store_scatter(gchunk, [row16 + hh], g16 * r16)
                return 0

            lax.fori_loop(0, nblk, blk, 0)
            pltpu.sync_copy(
                gchunk,
                al_hbm.at[wid, 0, pl.ds(cc * CB * CH * NHEADS,
                                        CB * CH * NHEADS)])

    f = pl.kernel(
        body,
        compiler_params=pltpu.CompilerParams(needs_layout_passes=False),
        out_type=pltpu.HBM((NW, 1, NCHUNK * CB * CH * NHEADS), jnp.float32),
        mesh=_MESH,
        scratch_types=[
            pltpu.VMEM((CB * CH,), jnp.int32),
            pltpu.VMEM((NN * 2 * NHEADS,), jnp.float32),
            pltpu.VMEM((CB * CH * NHEADS,), jnp.float32),
        ],
    )
    return f(g4, r8, idx_s)


def _sc_aggregate(h_d, al4, idx_g, idx_s):
    CB = 16
    NCHUNK = (BPW + CB - 1) // CB

    def body(h_hbm, al_hbm, ig_hbm, is_hbm,
             out_hbm,
             ig_v, is_v, achunk, rows, zbuf, out_sh):
        c = lax.axis_index("c")
        s = lax.axis_index("s")
        wid = c * NS + s

        _zero_rows(zbuf, 8, 8)
        for t in range(NPN // 8):
            pltpu.sync_copy(zbuf, out_sh.at[pl.ds(s * NPN + t * 8, 8)])
        plsc.subcore_barrier()

        pltpu.sync_copy(ig_hbm.at[wid], ig_v)
        pltpu.sync_copy(is_hbm.at[wid], is_v)

        for cc in range(NCHUNK):
            nblk = min(CB, BPW - cc * CB)
            pltpu.sync_copy(
                al_hbm.at[wid, 0, pl.ds(cc * CB * CH * NHEADS,
                                        CB * CH * NHEADS)], achunk)

            def blk(j2, _):
                j = cc * CB + j2
                pltpu.sync_copy(h_hbm.at[ig_v.at[j]], rows)

                def edge(k, _):
                    for hh in range(NHEADS):
                        av = plsc.load_gather(
                            achunk,
                            [jnp.full((16,),
                                      j2 * CH * NHEADS + k * NHEADS + hh,
                                      jnp.int32)])
                        q0 = rows[k, pl.ds(hh * DHEAD, 16)]
                        q1 = rows[k, pl.ds(hh * DHEAD + 16, 16)]
                        rows[k, pl.ds(hh * DHEAD, 16)] = q0 * av
                        rows[k, pl.ds(hh * DHEAD + 16, 16)] = q1 * av
                    return 0

                lax.fori_loop(0, CH, edge, 0)
                pltpu.sync_copy(rows, out_sh.at[is_v.at[j]], add=True)
                return 0

            lax.fori_loop(0, nblk, blk, 0)

        plsc.subcore_barrier()
        for t in range(NPN // 8):
            sl = pl.ds(s * NPN + t * 8, 8)
            pltpu.sync_copy(out_sh.at[sl], out_hbm.at[c, sl])

    f = pl.kernel(
        body,
        compiler_params=pltpu.CompilerParams(needs_layout_passes=False),
        out_type=pltpu.HBM((NC, NNP, HH), jnp.float32),
        mesh=_MESH,
        scratch_types=[
            pltpu.VMEM((BPW, CH), jnp.int32),
            pltpu.VMEM((BPW, CH), jnp.int32),
            pltpu.VMEM((CB * CH * NHEADS,), jnp.float32),
            pltpu.VMEM((CH, HH), jnp.float32),
            pltpu.VMEM((8, HH), jnp.float32),
            pltpu.VMEM_SHARED((NNP, HH), jnp.float32),
        ],
    )
    return f(h_d, al4, idx_g, idx_s)


_RB = 400


def _tc_round(feats, wcat, mai, mbi, mao, mbo):
    nf = len(feats)

    def body(*refs):
        frefs = refs[:nf]
        w_ref, mai_ref, mbi_ref, mao_ref, mbo_ref = refs[nf:nf + 5]
        fea_ref, h_ref, ai_ref, bi_ref, ao_ref, bo_ref = refs[nf + 5:]
        fea = frefs[0][...]
        for fr in frefs[1:]:
            fea = fea + fr[...]
        fea_ref[...] = fea
        h = jnp.dot(fea, w_ref[...], preferred_element_type=jnp.float32)
        h_ref[...] = h
        ai_ref[...] = jnp.dot(h, mai_ref[...], preferred_element_type=jnp.float32)
        bi_ref[...] = jnp.dot(h, mbi_ref[...], preferred_element_type=jnp.float32)
        ao_ref[...] = jnp.dot(h, mao_ref[...], preferred_element_type=jnp.float32)
        bo_ref[...] = jnp.dot(h, mbo_ref[...], preferred_element_type=jnp.float32)

    fspec = pl.BlockSpec((_RB, HH), lambda i: (i, 0))
    mspec = pl.BlockSpec((2 * HH, NHEADS), lambda i: (0, 0))
    sspec = pl.BlockSpec((_RB, NHEADS), lambda i: (i, 0))
    return pl.pallas_call(
        body,
        grid=(NN // _RB,),
        in_specs=[fspec] * nf + [
            pl.BlockSpec((HH, 2 * HH), lambda i: (0, 0)),
            mspec, mspec, mspec, mspec,
        ],
        out_specs=[
            pl.BlockSpec((_RB, HH), lambda i: (i, 0)),
            pl.BlockSpec((_RB, 2 * HH), lambda i: (i, 0)),
            sspec, sspec, sspec, sspec,
        ],
        out_shape=[
            jax.ShapeDtypeStruct((NN, HH), jnp.float32),
            jax.ShapeDtypeStruct((NN, 2 * HH), jnp.float32),
            jax.ShapeDtypeStruct((NN, NHEADS), jnp.float32),
            jax.ShapeDtypeStruct((NN, NHEADS), jnp.float32),
            jax.ShapeDtypeStruct((NN, NHEADS), jnp.float32),
            jax.ShapeDtypeStruct((NN, NHEADS), jnp.float32),
        ],
    )(*feats, wcat, mai, mbi, mao, mbo)


def _tc_recip(den_i, den_o):
    def body(di_ref, do_ref, r_ref):
        di = jnp.sum(di_ref[...], axis=(0, 1))
        do_ = jnp.sum(do_ref[...], axis=(0, 1))
        r_ref[...] = jnp.concatenate(
            [1.0 / (di + 1e-9), 1.0 / (do_ + 1e-9)], axis=-1)

    dspec = pl.BlockSpec((NC, NS, _RB, NHEADS), lambda i: (0, 0, i, 0))
    return pl.pallas_call(
        body,
        grid=(NN // _RB,),
        in_specs=[dspec, dspec],
        out_specs=pl.BlockSpec((_RB, 2 * NHEADS), lambda i: (i, 0)),
        out_shape=jax.ShapeDtypeStruct((NN, 2 * NHEADS), jnp.float32),
    )(den_i, den_o)


def _tc_final(parts, wm, bm2):
    np_ = len(parts)

    def body(*refs):
        prefs = refs[:np_]
        wm_ref, bm_ref = refs[np_:np_ + 2]
        logits_ref, logp_ref = refs[np_ + 2:np_ + 4]
        acc_ref = refs[np_ + 4]
        i = pl.program_id(0)
        bf = prefs[0][...]
        for pr in prefs[1:]:
            bf = bf + pr[...]
        blkmax = jnp.max(bf, axis=0, keepdims=True)

        @pl.when(i == 0)
        def _():
            acc_ref[...] = blkmax

        @pl.when(i > 0)
        def _():
            acc_ref[...] = jnp.maximum(acc_ref[...], blkmax)

        @pl.when(i == NN // _RB - 1)
        def _():
            pooled = acc_ref[...]
            logits = jnp.dot(pooled, wm_ref[...],
                             preferred_element_type=jnp.float32) + bm_ref[...]
            m = jnp.max(logits, axis=-1, keepdims=True)
            lse = m + jnp.log(jnp.sum(jnp.exp(logits - m), axis=-1,
                                      keepdims=True))
            logits_ref[...] = logits
            logp_ref[...] = logits - lse

    pspec = pl.BlockSpec((_RB, HH), lambda i: (i, 0))
    return pl.pallas_call(
        body,
        grid=(NN // _RB,),
        in_specs=[pspec] * np_ + [
            pl.BlockSpec((HH, 104), lambda i: (0, 0)),
            pl.BlockSpec((1, 104), lambda i: (0, 0)),
        ],
        out_specs=[
            pl.BlockSpec((1, 104), lambda i: (0, 0)),
            pl.BlockSpec((1, 104), lambda i: (0, 0)),
        ],
        out_shape=[
            jax.ShapeDtypeStruct((1, 104), jnp.float32),
            jax.ShapeDtypeStruct((1, 104), jnp.float32),
        ],
        scratch_shapes=[pltpu.VMEM((1, HH), jnp.float32)],
    )(*parts, wm, bm2)


def _blockdiag(a, top):
    eye = jnp.eye(NHEADS, dtype=a.dtype)
    m = (a[:, :, None] * eye[:, None, :]).reshape(HH, NHEADS)
    z = jnp.zeros((HH, NHEADS), jnp.float32)
    return jnp.concatenate([m, z] if top else [z, m], axis=0)


def _round(feats, wi, asi, adi, wo, aso, ado, src3, dst3, reverse_o):
    wcat = jnp.concatenate([wi, wo], axis=1)
    mai = _blockdiag(asi, True)
    mbi = _blockdiag(adi, True)
    mao = _blockdiag(aso, False)
    mbo = _blockdiag(ado, False)
    fea, hcat, a_i, b_i, a_o, b_o = _tc_round(feats, wcat, mai, mbi, mao, mbo)
    h_i = hcat[:, :HH]
    h_o = hcat[:, HH:]
    srcF = jnp.pad(src3.reshape(NW, BPW * CH),
                   ((0, 0), (0, 240))).reshape(NW, 1, BPW * CH + 240)
    dstF = jnp.pad(dst3.reshape(NW, BPW * CH),
                   ((0, 0), (0, 240))).reshape(NW, 1, BPW * CH + 240)
    den_i, g_i = _sc_scores(a_i.reshape(NN * NHEADS), b_i.reshape(NN * NHEADS),
                            srcF, dstF)
    if reverse_o:
        den_o, g_o = _sc_scores(a_o.reshape(NN * NHEADS),
                                b_o.reshape(NN * NHEADS), dstF, srcF)
    else:
        den_o, g_o = _sc_scores(a_o.reshape(NN * NHEADS),
                                b_o.reshape(NN * NHEADS), srcF, dstF)
    r8 = _tc_recip(
        den_i[:, :, 0, :NN * NHEADS].reshape(NC, NS, NN, NHEADS),
        den_o[:, :, 0, :NN * NHEADS].reshape(NC, NS, NN, NHEADS),
    ).reshape(NN * 2 * NHEADS)
    al_i = _sc_alpha(g_i, r8, dstF, 0)
    gi = _sc_aggregate(h_i, al_i, src3, dst3)
    if reverse_o:
        al_o = _sc_alpha(g_o, r8, srcF, NHEADS)
        go = _sc_aggregate(h_o, al_o, dst3, src3)
    else:
        al_o = _sc_alpha(g_o, r8, dstF, NHEADS)
        go = _sc_aggregate(h_o, al_o, src3, dst3)
    return fea, gi, go


def kernel(x, edge_index, label, Wi1, asi1, adi1, Wo1, aso1, ado1, Wi2, asi2, adi2, Wo2, aso2, ado2, Wi3, asi3, adi3, Wo3, aso3, ado3, Wm, bm):
    src3 = edge_index[0].reshape(NW, BPW, CH)
    dst3 = edge_index[1].reshape(NW, BPW, CH)

    fea1, gi1, go1 = _round([x], Wi1, asi1, adi1, Wo1, aso1, ado1,
                            src3, dst3, True)
    p1 = [gi1[0, :NN], gi1[1, :NN], go1[0, :NN], go1[1, :NN], fea1]
    fea2, gi2, go2 = _round(p1, Wi2, asi2, adi2, Wo2, aso2, ado2,
                            src3, dst3, False)
    p2 = [gi2[0, :NN], gi2[1, :NN], go2[0, :NN], go2[1, :NN], fea2]
    fea3, gi3, go3 = _round(p2, Wi3, asi3, adi3, Wo3, aso3, ado3,
                            src3, dst3, False)
    p3 = [gi3[0, :NN], gi3[1, :NN], go3[0, :NN], go3[1, :NN], fea3]

    logits, logp = _tc_final(p3, Wm, bm.reshape(1, 104))
    loss = -logp[0, label[0]]
    return (logits, loss)
